# baseline (speedup 1.0000x reference)
import functools

import jax
import jax.numpy as jnp
from jax import lax
from jax.experimental import pallas as pl
from jax.experimental.pallas import tpu as pltpu
from jax.experimental.pallas import tpu_sc as plsc

N_OUT = 100_000
NPAD = 102_400
PC = NPAD // 16
CH = 4_000


def _prep_body(coor_ref, edges_ref, p_ref, q_ref, b_ref, v_ref, seg_ref):
    BV = v_ref.shape[0]
    x = coor_ref[...].reshape(BV, 768)
    v = jax.lax.dot_general(
        x, p_ref[...], (((1,), (0,)), ((), ())),
        preferred_element_type=jnp.float32,
        precision=jax.lax.Precision.HIGHEST,
    )
    v_ref[...] = v + b_ref[0]
    e = edges_ref[...].astype(jnp.float32).reshape(BV, 256)
    s = jax.lax.dot_general(
        e, q_ref[...], (((1,), (0,)), ((), ())),
        preferred_element_type=jnp.float32,
        precision=jax.lax.Precision.HIGHEST,
    )
    seg_ref[...] = s.astype(jnp.int32)


def _prep(coor, edges, W1, b1, E, BE):
    P = jnp.kron(jnp.eye(128, dtype=jnp.float32), W1[0][:, None])
    Q = jnp.kron(jnp.eye(128, dtype=jnp.float32),
                 jnp.array([[0.0], [1.0]], dtype=jnp.float32))
    coorm = coor.reshape(E * 6 // 128, 128)
    edgesm = edges.reshape(E * 2 // 128, 128)
    BV = BE // 128
    grid = (E // BE,)
    v, seg = pl.pallas_call(
        _prep_body,
        grid=grid,
        in_specs=[
            pl.BlockSpec((BV * 6, 128), lambda i: (i, 0)),
            pl.BlockSpec((BV * 2, 128), lambda i: (i, 0)),
            pl.BlockSpec((768, 128), lambda i: (0, 0)),
            pl.BlockSpec((256, 128), lambda i: (0, 0)),
            pl.BlockSpec(memory_space=pltpu.SMEM),
        ],
        out_specs=[
            pl.BlockSpec((BV, 128), lambda i: (i, 0)),
            pl.BlockSpec((BV, 128), lambda i: (i, 0)),
        ],
        out_shape=[
            jax.ShapeDtypeStruct((E // 128, 128), jnp.float32),
            jax.ShapeDtypeStruct((E // 128, 128), jnp.int32),
        ],
    )(coorm, edgesm, P, Q, b1)
    return v.reshape(E), seg.reshape(E)


def _make_scatter(E):
    EW = E // 32
    NCH = EW // CH
    mesh = plsc.VectorSubcoreMesh(core_axis_name="c", subcore_axis_name="s")

    @functools.partial(
        pl.kernel,
        out_type=[
            jax.ShapeDtypeStruct((2, NPAD), jnp.float32),
            jax.ShapeDtypeStruct((2, NPAD), jnp.float32),
        ],
        mesh=mesh,
        scratch_types=[
            pltpu.VMEM((CH,), jnp.float32),
            pltpu.VMEM((CH,), jnp.int32),
            pltpu.VMEM((CH,), jnp.float32),
            pltpu.VMEM_SHARED((NPAD,), jnp.float32),
            pltpu.VMEM_SHARED((NPAD,), jnp.float32),
        ],
    )
    def scatter(v_h, seg_h, zeros_h, ones_h, sums_out, cnt_out,
                vv, dv, ones_v, sums_sh, cnt_sh):
        cid = lax.axis_index("c")
        sid = lax.axis_index("s")
        wid = cid * 16 + sid
        pltpu.sync_copy(zeros_h, sums_sh.at[pl.ds(sid * PC, PC)])
        pltpu.sync_copy(zeros_h, cnt_sh.at[pl.ds(sid * PC, PC)])
        pltpu.sync_copy(ones_h, ones_v)
        plsc.subcore_barrier()

        def step(k, carry):
            off = pl.multiple_of(wid * EW + k * CH, 8)
            pltpu.sync_copy(v_h.at[pl.ds(off, CH)], vv)
            pltpu.sync_copy(seg_h.at[pl.ds(off, CH)], dv)
            pltpu.sync_copy(vv, sums_sh.at[dv], add=True)
            pltpu.sync_copy(ones_v, cnt_sh.at[dv], add=True)
            return carry

        lax.fori_loop(0, NCH, step, 0)
        plsc.subcore_barrier()
        pltpu.sync_copy(sums_sh.at[pl.ds(sid * PC, PC)],
                        sums_out.at[cid, pl.ds(sid * PC, PC)])
        pltpu.sync_copy(cnt_sh.at[pl.ds(sid * PC, PC)],
                        cnt_out.at[cid, pl.ds(sid * PC, PC)])

    return scatter


def _fin_body(s_ref, c_ref, o_ref):
    s = s_ref[0] + s_ref[1]
    c = c_ref[0] + c_ref[1]
    o_ref[...] = s / jnp.maximum(c, 1.0)


def _finalize(sums, cnts, BN=10_240):
    grid = (NPAD // BN,)
    return pl.pallas_call(
        _fin_body,
        grid=grid,
        in_specs=[
            pl.BlockSpec((2, BN), lambda i: (0, i)),
            pl.BlockSpec((2, BN), lambda i: (0, i)),
        ],
        out_specs=pl.BlockSpec((BN,), lambda i: (i,)),
        out_shape=jax.ShapeDtypeStruct((NPAD,), jnp.float32),
    )(sums, cnts)


def kernel(edges, coor, W1, b1, W2, b2, W4, b4):
    E = coor.shape[1]
    BE = 25_600
    v, seg = _prep(coor, edges, W1, b1, E, BE)
    zeros_h = jnp.zeros((PC,), jnp.float32)
    ones_h = jnp.ones((CH,), jnp.float32)
    sums, cnts = _make_scatter(E)(v, seg, zeros_h, ones_h)
    mean_pad = _finalize(sums, cnts)
    return mean_pad[:N_OUT][None, :, None]


# speedup vs baseline: 23.0285x; 23.0285x over previous
import functools

import jax
import jax.numpy as jnp
from jax import lax
from jax.experimental import pallas as pl
from jax.experimental.pallas import tpu as pltpu
from jax.experimental.pallas import tpu_sc as plsc

N_OUT = 100_000
NPAD = 102_400
PC = NPAD // 16
CH = 4_000


def _prep_body(coor_ref, edges_ref, w_ref, b_ref, v_ref, seg_ref):
    x = coor_ref[...]
    v = x[0] * w_ref[0, 0]
    for k in range(1, 6):
        v = v + x[k] * w_ref[0, k]
    v_ref[...] = v + b_ref[0]
    seg_ref[...] = edges_ref[1]


def _prep(coor, edges, W1, b1, E, BE):
    coorT = jnp.transpose(coor[0])
    edgesT = jnp.transpose(edges[0])
    grid = (E // BE,)
    return pl.pallas_call(
        _prep_body,
        grid=grid,
        in_specs=[
            pl.BlockSpec((6, BE), lambda i: (0, i)),
            pl.BlockSpec((2, BE), lambda i: (0, i)),
            pl.BlockSpec(memory_space=pltpu.SMEM),
            pl.BlockSpec(memory_space=pltpu.SMEM),
        ],
        out_specs=[
            pl.BlockSpec((BE,), lambda i: (i,)),
            pl.BlockSpec((BE,), lambda i: (i,)),
        ],
        out_shape=[
            jax.ShapeDtypeStruct((E,), jnp.float32),
            jax.ShapeDtypeStruct((E,), jnp.int32),
        ],
    )(coorT, edgesT, W1, b1)


def _make_scatter(E):
    EW = E // 32
    NCH = EW // CH
    mesh = plsc.VectorSubcoreMesh(core_axis_name="c", subcore_axis_name="s")

    @functools.partial(
        pl.kernel,
        out_type=[
            jax.ShapeDtypeStruct((2, NPAD), jnp.float32),
            jax.ShapeDtypeStruct((2, NPAD), jnp.float32),
        ],
        mesh=mesh,
        scratch_types=[
            pltpu.VMEM((CH,), jnp.float32),
            pltpu.VMEM((CH,), jnp.int32),
            pltpu.VMEM((CH,), jnp.float32),
            pltpu.VMEM_SHARED((NPAD,), jnp.float32),
            pltpu.VMEM_SHARED((NPAD,), jnp.float32),
        ],
    )
    def scatter(v_h, seg_h, zeros_h, ones_h, sums_out, cnt_out,
                vv, dv, ones_v, sums_sh, cnt_sh):
        cid = lax.axis_index("c")
        sid = lax.axis_index("s")
        wid = cid * 16 + sid
        pltpu.sync_copy(zeros_h, sums_sh.at[pl.ds(sid * PC, PC)])
        pltpu.sync_copy(zeros_h, cnt_sh.at[pl.ds(sid * PC, PC)])
        pltpu.sync_copy(ones_h, ones_v)
        plsc.subcore_barrier()

        def step(k, carry):
            off = pl.multiple_of(wid * EW + k * CH, 8)
            pltpu.sync_copy(v_h.at[pl.ds(off, CH)], vv)
            pltpu.sync_copy(seg_h.at[pl.ds(off, CH)], dv)
            pltpu.sync_copy(vv, sums_sh.at[dv], add=True)
            pltpu.sync_copy(ones_v, cnt_sh.at[dv], add=True)
            return carry

        lax.fori_loop(0, NCH, step, 0)
        plsc.subcore_barrier()
        pltpu.sync_copy(sums_sh.at[pl.ds(sid * PC, PC)],
                        sums_out.at[cid, pl.ds(sid * PC, PC)])
        pltpu.sync_copy(cnt_sh.at[pl.ds(sid * PC, PC)],
                        cnt_out.at[cid, pl.ds(sid * PC, PC)])

    return scatter


def _fin_body(s_ref, c_ref, o_ref):
    s = s_ref[0] + s_ref[1]
    c = c_ref[0] + c_ref[1]
    o_ref[...] = s / jnp.maximum(c, 1.0)


def _finalize(sums, cnts, BN=10_240):
    grid = (NPAD // BN,)
    return pl.pallas_call(
        _fin_body,
        grid=grid,
        in_specs=[
            pl.BlockSpec((2, BN), lambda i: (0, i)),
            pl.BlockSpec((2, BN), lambda i: (0, i)),
        ],
        out_specs=pl.BlockSpec((BN,), lambda i: (i,)),
        out_shape=jax.ShapeDtypeStruct((NPAD,), jnp.float32),
    )(sums, cnts)


def kernel(edges, coor, W1, b1, W2, b2, W4, b4):
    E = coor.shape[1]
    BE = 128_000
    v, seg = _prep(coor, edges, W1, b1, E, BE)
    zeros_h = jnp.zeros((PC,), jnp.float32)
    ones_h = jnp.ones((CH,), jnp.float32)
    sums, cnts = _make_scatter(E)(v, seg, zeros_h, ones_h)
    mean_pad = _finalize(sums, cnts)
    return mean_pad[:N_OUT][None, :, None]


# speedup vs baseline: 27.1494x; 1.1790x over previous
import functools

import jax
import jax.numpy as jnp
from jax import lax
from jax.experimental import pallas as pl
from jax.experimental.pallas import tpu as pltpu
from jax.experimental.pallas import tpu_sc as plsc

N_OUT = 100_000
NPAD = 102_400
PC = NPAD // 16
CH = 5_000


def _prep_body(coor_ref, edges_ref, w_ref, b_ref, v_ref, seg_ref):
    x = coor_ref[...]
    v = x[0] * w_ref[0, 0]
    for k in range(1, 6):
        v = v + x[k] * w_ref[0, k]
    v_ref[...] = v + b_ref[0]
    seg_ref[...] = edges_ref[1]


def _prep(coor, edges, W1, b1, E, BE):
    coorT = jnp.transpose(coor[0])
    edgesT = jnp.transpose(edges[0])
    grid = (E // BE,)
    return pl.pallas_call(
        _prep_body,
        grid=grid,
        in_specs=[
            pl.BlockSpec((6, BE), lambda i: (0, i)),
            pl.BlockSpec((2, BE), lambda i: (0, i)),
            pl.BlockSpec(memory_space=pltpu.SMEM),
            pl.BlockSpec(memory_space=pltpu.SMEM),
        ],
        out_specs=[
            pl.BlockSpec((BE,), lambda i: (i,)),
            pl.BlockSpec((BE,), lambda i: (i,)),
        ],
        out_shape=[
            jax.ShapeDtypeStruct((E,), jnp.float32),
            jax.ShapeDtypeStruct((E,), jnp.int32),
        ],
    )(coorT, edgesT, W1, b1)


def _make_scatter(E):
    EW = E // 32
    NCH = EW // CH
    mesh = plsc.VectorSubcoreMesh(core_axis_name="c", subcore_axis_name="s")

    @functools.partial(
        pl.kernel,
        out_type=[
            jax.ShapeDtypeStruct((2, NPAD), jnp.float32),
            jax.ShapeDtypeStruct((2, NPAD), jnp.float32),
        ],
        mesh=mesh,
        scratch_types=[
            pltpu.VMEM((CH,), jnp.float32),
            pltpu.VMEM((CH,), jnp.int32),
            pltpu.VMEM((CH,), jnp.float32),
            pltpu.VMEM((CH,), jnp.int32),
            pltpu.VMEM((CH,), jnp.float32),
            pltpu.VMEM_SHARED((NPAD,), jnp.float32),
            pltpu.VMEM_SHARED((NPAD,), jnp.float32),
            pltpu.SemaphoreType.DMA,
            pltpu.SemaphoreType.DMA,
            pltpu.SemaphoreType.DMA,
            pltpu.SemaphoreType.DMA,
            pltpu.SemaphoreType.DMA,
            pltpu.SemaphoreType.DMA,
            pltpu.SemaphoreType.DMA,
            pltpu.SemaphoreType.DMA,
        ],
    )
    def scatter(v_h, seg_h, zeros_h, ones_h, sums_out, cnt_out,
                vv0, dv0, vv1, dv1, ones_v, sums_sh, cnt_sh,
                sv0, sd0, sv1, sd1, ss0, sc0, ss1, sc1):
        cid = lax.axis_index("c")
        sid = lax.axis_index("s")
        wid = cid * 16 + sid
        pltpu.sync_copy(zeros_h, sums_sh.at[pl.ds(sid * PC, PC)])
        pltpu.sync_copy(zeros_h, cnt_sh.at[pl.ds(sid * PC, PC)])
        pltpu.sync_copy(ones_h, ones_v)
        plsc.subcore_barrier()

        vv = (vv0, vv1)
        dv = (dv0, dv1)
        sv = (sv0, sv1)
        sd = (sd0, sd1)
        ss = (ss0, ss1)
        sc = (sc0, sc1)

        def start_in(k, b):
            off = pl.multiple_of(wid * EW + k * CH, 8)
            pltpu.async_copy(v_h.at[pl.ds(off, CH)], vv[b], sv[b])
            pltpu.async_copy(seg_h.at[pl.ds(off, CH)], dv[b], sd[b])

        def wait_in(b):
            pltpu.make_async_copy(v_h.at[pl.ds(0, CH)], vv[b], sv[b]).wait()
            pltpu.make_async_copy(seg_h.at[pl.ds(0, CH)], dv[b], sd[b]).wait()

        def start_scat(b):
            pltpu.async_copy(vv[b], sums_sh.at[dv[b]], ss[b], add=True)
            pltpu.async_copy(ones_v, cnt_sh.at[dv[b]], sc[b], add=True)

        def wait_scat(b):
            pltpu.make_async_copy(vv[b], sums_sh.at[dv[b]], ss[b]).wait()
            pltpu.make_async_copy(ones_v, cnt_sh.at[dv[b]], sc[b]).wait()

        start_in(0, 0)

        def body(m, carry):
            k0 = 2 * m
            wait_in(0)
            start_scat(0)
            start_in(k0 + 1, 1)
            wait_scat(0)
            wait_in(1)
            start_scat(1)

            @pl.when(m < NCH // 2 - 1)
            def _():
                start_in(k0 + 2, 0)

            wait_scat(1)
            return carry

        lax.fori_loop(0, NCH // 2, body, 0)
        plsc.subcore_barrier()
        pltpu.sync_copy(sums_sh.at[pl.ds(sid * PC, PC)],
                        sums_out.at[cid, pl.ds(sid * PC, PC)])
        pltpu.sync_copy(cnt_sh.at[pl.ds(sid * PC, PC)],
                        cnt_out.at[cid, pl.ds(sid * PC, PC)])

    return scatter


def _fin_body(s_ref, c_ref, o_ref):
    s = s_ref[0] + s_ref[1]
    c = c_ref[0] + c_ref[1]
    o_ref[...] = s / jnp.maximum(c, 1.0)


def _finalize(sums, cnts, BN=10_240):
    grid = (NPAD // BN,)
    return pl.pallas_call(
        _fin_body,
        grid=grid,
        in_specs=[
            pl.BlockSpec((2, BN), lambda i: (0, i)),
            pl.BlockSpec((2, BN), lambda i: (0, i)),
        ],
        out_specs=pl.BlockSpec((BN,), lambda i: (i,)),
        out_shape=jax.ShapeDtypeStruct((NPAD,), jnp.float32),
    )(sums, cnts)


def kernel(edges, coor, W1, b1, W2, b2, W4, b4):
    E = coor.shape[1]
    BE = 128_000
    v, seg = _prep(coor, edges, W1, b1, E, BE)
    zeros_h = jnp.zeros((PC,), jnp.float32)
    ones_h = jnp.ones((CH,), jnp.float32)
    sums, cnts = _make_scatter(E)(v, seg, zeros_h, ones_h)
    mean_pad = _finalize(sums, cnts)
    return mean_pad[:N_OUT][None, :, None]


# speedup vs baseline: 30.6838x; 1.1302x over previous
import functools

import jax
import jax.numpy as jnp
from jax import lax
from jax.experimental import pallas as pl
from jax.experimental.pallas import tpu as pltpu
from jax.experimental.pallas import tpu_sc as plsc

N_OUT = 100_000
NPAD = 102_400
PC = NPAD // 16
CH = 5_000


def _prep_body(coor_ref, edges_ref, w_ref, b_ref, v_ref, seg_ref):
    x = coor_ref[...]
    v = x[0] * w_ref[0, 0]
    for k in range(1, 6):
        v = v + x[k] * w_ref[0, k]
    v_ref[...] = v + b_ref[0]
    seg_ref[...] = edges_ref[1]


def _prep(coorT, edgesT, W1, b1, EH, BE, base_blk):
    grid = (EH // BE,)
    return pl.pallas_call(
        _prep_body,
        grid=grid,
        in_specs=[
            pl.BlockSpec((6, BE), lambda i: (0, i + base_blk)),
            pl.BlockSpec((2, BE), lambda i: (0, i + base_blk)),
            pl.BlockSpec(memory_space=pltpu.SMEM),
            pl.BlockSpec(memory_space=pltpu.SMEM),
        ],
        out_specs=[
            pl.BlockSpec((BE,), lambda i: (i,)),
            pl.BlockSpec((BE,), lambda i: (i,)),
        ],
        out_shape=[
            jax.ShapeDtypeStruct((EH,), jnp.float32),
            jax.ShapeDtypeStruct((EH,), jnp.int32),
        ],
    )(coorT, edgesT, W1, b1)


def _make_scatter(E):
    EW = E // 32
    NCH = EW // CH
    mesh = plsc.VectorSubcoreMesh(core_axis_name="c", subcore_axis_name="s")

    @functools.partial(
        pl.kernel,
        out_type=[
            jax.ShapeDtypeStruct((2, NPAD), jnp.float32),
            jax.ShapeDtypeStruct((2, NPAD), jnp.float32),
        ],
        mesh=mesh,
        scratch_types=[
            pltpu.VMEM((CH,), jnp.float32),
            pltpu.VMEM((CH,), jnp.int32),
            pltpu.VMEM((CH,), jnp.float32),
            pltpu.VMEM((CH,), jnp.int32),
            pltpu.VMEM((CH,), jnp.float32),
            pltpu.VMEM_SHARED((NPAD,), jnp.float32),
            pltpu.VMEM_SHARED((NPAD,), jnp.float32),
            pltpu.SemaphoreType.DMA,
            pltpu.SemaphoreType.DMA,
            pltpu.SemaphoreType.DMA,
            pltpu.SemaphoreType.DMA,
            pltpu.SemaphoreType.DMA,
            pltpu.SemaphoreType.DMA,
            pltpu.SemaphoreType.DMA,
            pltpu.SemaphoreType.DMA,
        ],
    )
    def scatter(v_h, seg_h, zeros_h, ones_h, sums_out, cnt_out,
                vv0, dv0, vv1, dv1, ones_v, sums_sh, cnt_sh,
                sv0, sd0, sv1, sd1, ss0, sc0, ss1, sc1):
        cid = lax.axis_index("c")
        sid = lax.axis_index("s")
        wid = cid * 16 + sid
        pltpu.sync_copy(zeros_h, sums_sh.at[pl.ds(sid * PC, PC)])
        pltpu.sync_copy(zeros_h, cnt_sh.at[pl.ds(sid * PC, PC)])
        pltpu.sync_copy(ones_h, ones_v)
        plsc.subcore_barrier()

        vv = (vv0, vv1)
        dv = (dv0, dv1)
        sv = (sv0, sv1)
        sd = (sd0, sd1)
        ss = (ss0, ss1)
        sc = (sc0, sc1)

        def start_in(k, b):
            off = pl.multiple_of(wid * EW + k * CH, 8)
            pltpu.async_copy(v_h.at[pl.ds(off, CH)], vv[b], sv[b])
            pltpu.async_copy(seg_h.at[pl.ds(off, CH)], dv[b], sd[b])

        def wait_in(b):
            pltpu.make_async_copy(v_h.at[pl.ds(0, CH)], vv[b], sv[b]).wait()
            pltpu.make_async_copy(seg_h.at[pl.ds(0, CH)], dv[b], sd[b]).wait()

        def start_scat(b):
            pltpu.async_copy(vv[b], sums_sh.at[dv[b]], ss[b], add=True)
            pltpu.async_copy(ones_v, cnt_sh.at[dv[b]], sc[b], add=True)

        def wait_scat(b):
            pltpu.make_async_copy(vv[b], sums_sh.at[dv[b]], ss[b]).wait()
            pltpu.make_async_copy(ones_v, cnt_sh.at[dv[b]], sc[b]).wait()

        start_in(0, 0)

        def body(m, carry):
            k0 = 2 * m
            wait_in(0)
            start_scat(0)
            start_in(k0 + 1, 1)
            wait_scat(0)
            wait_in(1)
            start_scat(1)

            @pl.when(m < NCH // 2 - 1)
            def _():
                start_in(k0 + 2, 0)

            wait_scat(1)
            return carry

        lax.fori_loop(0, NCH // 2, body, 0)
        plsc.subcore_barrier()
        pltpu.sync_copy(sums_sh.at[pl.ds(sid * PC, PC)],
                        sums_out.at[cid, pl.ds(sid * PC, PC)])
        pltpu.sync_copy(cnt_sh.at[pl.ds(sid * PC, PC)],
                        cnt_out.at[cid, pl.ds(sid * PC, PC)])

    return scatter


def _fin_body(s0_ref, c0_ref, s1_ref, c1_ref, o_ref):
    s = s0_ref[0] + s0_ref[1] + s1_ref[0] + s1_ref[1]
    c = c0_ref[0] + c0_ref[1] + c1_ref[0] + c1_ref[1]
    o_ref[...] = s / jnp.maximum(c, 1.0)


def _finalize(s0, c0, s1, c1, BN=10_240):
    grid = (NPAD // BN,)
    spec = pl.BlockSpec((2, BN), lambda i: (0, i))
    return pl.pallas_call(
        _fin_body,
        grid=grid,
        in_specs=[spec, spec, spec, spec],
        out_specs=pl.BlockSpec((BN,), lambda i: (i,)),
        out_shape=jax.ShapeDtypeStruct((NPAD,), jnp.float32),
    )(s0, c0, s1, c1)


def kernel(edges, coor, W1, b1, W2, b2, W4, b4):
    E = coor.shape[1]
    BE = 128_000
    EH = E // 2
    coorT = jnp.transpose(coor[0])
    edgesT = jnp.transpose(edges[0])
    zeros_h = jnp.zeros((PC,), jnp.float32)
    ones_h = jnp.ones((CH,), jnp.float32)
    scat = _make_scatter(EH)
    v0, seg0 = _prep(coorT, edgesT, W1, b1, EH, BE, 0)
    s0, c0 = scat(v0, seg0, zeros_h, ones_h)
    v1, seg1 = _prep(coorT, edgesT, W1, b1, EH, BE, EH // BE)
    s1, c1 = scat(v1, seg1, zeros_h, ones_h)
    mean_pad = _finalize(s0, c0, s1, c1)
    return mean_pad[:N_OUT][None, :, None]
